# baseline (speedup 1.0000x reference)
import functools

import jax
import jax.numpy as jnp
import numpy as np
from jax import lax
from jax.experimental import pallas as pl
from jax.experimental.pallas import tpu as pltpu
from jax.experimental.pallas import tpu_sc as plsc


def _conv(x, w, stride=1, padding=0, dilation=1):
    return lax.conv_general_dilated(
        x, w, (stride, stride), [(padding, padding), (padding, padding)],
        rhs_dilation=(dilation, dilation),
        dimension_numbers=('NCHW', 'OIHW', 'NCHW'))


def _bn(x, p):
    g, b, m, v = p
    inv = g / jnp.sqrt(v + 1e-5)
    return x * inv[None, :, None, None] + (b - m * inv)[None, :, None, None]


def _resblock(x, p, stride=1, first_dilation=1, dilation=1):
    b2 = jax.nn.relu(_bn(x, p['bn2a']))
    xbr = b2
    branch1 = _conv(b2, p['w1'], stride) if 'w1' in p else x
    b2 = _conv(b2, p['w2a'], stride, padding=first_dilation, dilation=first_dilation)
    b2 = jax.nn.relu(_bn(b2, p['bn2b1']))
    b2 = _conv(b2, p['w2b1'], 1, padding=dilation, dilation=dilation)
    return branch1 + b2, xbr


def _resblock_bot(x, p, dilation=1):
    b2 = jax.nn.relu(_bn(x, p['bn2a']))
    xbr = b2
    branch1 = _conv(b2, p['w1'], 1)
    b2 = _conv(b2, p['w2a'], 1)
    b2 = jax.nn.relu(_bn(b2, p['bn2b1']))
    b2 = _conv(b2, p['w2b1'], 1, padding=dilation, dilation=dilation)
    b2 = jax.nn.relu(_bn(b2, p['bn2b2']))
    b2 = _conv(b2, p['w2b2'], 1)
    return branch1 + b2, xbr


def _backbone(x, p):
    h = _conv(x, p['conv1a'], 1, padding=1)
    h, _ = _resblock(h, p['b2'], stride=2)
    h, _ = _resblock(h, p['b2_1'])
    h, _ = _resblock(h, p['b2_2'])
    h, _ = _resblock(h, p['b3'], stride=2)
    h, _ = _resblock(h, p['b3_1'])
    h, _ = _resblock(h, p['b3_2'])
    h, _ = _resblock(h, p['b4'], stride=2)
    for name in ('b4_1', 'b4_2', 'b4_3', 'b4_4', 'b4_5'):
        h, _ = _resblock(h, p[name])
    h, conv4 = _resblock(h, p['b5'], stride=1, first_dilation=1, dilation=2)
    h, _ = _resblock(h, p['b5_1'], dilation=2)
    h, _ = _resblock(h, p['b5_2'], dilation=2)
    h, conv5 = _resblock_bot(h, p['b6'], dilation=4)
    h, _ = _resblock_bot(h, p['b7'], dilation=4)
    conv6 = jax.nn.relu(_bn(h, p['bn7']))
    f8_3 = jax.nn.elu(_conv(conv4, p['f8_3']))
    f8_4 = jax.nn.elu(_conv(conv5, p['f8_4']))
    f8_5 = jax.nn.elu(_conv(conv6, p['f8_5']))
    f = jax.nn.elu(_conv(jnp.concatenate([f8_3, f8_4, f8_5], axis=1), p['f9']))
    return f


_LANES = 16


def _shuffle(v, perm):
    return lax.gather(
        v, perm[:, None],
        dimension_numbers=lax.GatherDimensionNumbers(
            offset_dims=(), collapsed_slice_dims=(0,), start_index_map=(0,)),
        slice_sizes=(1,),
        mode=lax.GatherScatterMode.PROMISE_IN_BOUNDS)


@functools.lru_cache(maxsize=None)
def _make_affinity_head(hw, c, c_pad, d, np_pad):
    nc, ns = 2, 16
    nw = nc * ns
    ch = np_pad // nw
    nch = c // _LANES
    ng = ch // _LANES

    mesh = plsc.VectorSubcoreMesh(core_axis_name="c", subcore_axis_name="s")

    assert d % 2 == 0

    ra = (ng // 2 + 1) * _LANES
    rb = ch - ra

    @functools.partial(
        pl.kernel,
        out_type=jax.ShapeDtypeStruct((nw, d, ch), jnp.float32),
        mesh=mesh,
        scratch_types=[
            pltpu.VMEM((d, ch), jnp.int32),
            pltpu.VMEM((8, ch), jnp.int32),
            pltpu.VMEM((ch, c_pad), jnp.float32),
            pltpu.VMEM((ra, c_pad), jnp.float32),
            pltpu.VMEM((rb, c_pad), jnp.float32),
            pltpu.VMEM((d, ch), jnp.float32),
            pltpu.SemaphoreType.DMA,
            pltpu.SemaphoreType.DMA,
        ],
    )
    def head(fv_hbm, indf_hbm, indt_hbm, out_hbm,
             idx_v, fidx_v, ff_v, ft_a, ft_b, out_v, sem_a, sem_b):
        wid = lax.axis_index("s") * nc + lax.axis_index("c")
        pltpu.sync_copy(indt_hbm.at[wid], idx_v)
        pltpu.sync_copy(indf_hbm.at[wid], fidx_v.at[0])
        pltpu.async_copy(fv_hbm.at[fidx_v.at[0]], ff_v, sem_a).wait()
        lane = lax.iota(jnp.int32, _LANES)
        inv_c = jnp.float32(-1.0 / c)
        perms = [lane ^ jnp.int32(sh) for sh in (8, 4, 2, 1)]

        def compute(dd, ft_v, row0, ngrp):
            def g_body(g, _):
                def row_sum(row):
                    accs = [jnp.zeros((_LANES,), jnp.float32) for _ in range(4)]
                    for j in range(nch):
                        t = ft_v[row, pl.ds(j * _LANES, _LANES)]
                        f = ff_v[row0 + row, pl.ds(j * _LANES, _LANES)]
                        accs[j % 4] = accs[j % 4] + jnp.abs(t - f)
                    v = (accs[0] + accs[1]) + (accs[2] + accs[3])
                    for perm in perms:
                        v = v + _shuffle(v, perm)
                    return v

                def r_body(r2, rowsums):
                    r = r2 * 2
                    v0 = row_sum(g * _LANES + r)
                    v1 = row_sum(g * _LANES + r + 1)
                    rowsums = jnp.where(lane == r, v0, rowsums)
                    return jnp.where(lane == r + 1, v1, rowsums)

                rowsums = lax.fori_loop(
                    0, _LANES // 2, r_body, jnp.zeros((_LANES,), jnp.float32))
                out_v[dd, pl.ds(row0 + g * _LANES, _LANES)] = (
                    jnp.exp(rowsums * inv_c))
                return 0

            lax.fori_loop(0, ngrp, g_body, 0)

        def gather_a(dd):
            return pltpu.async_copy(
                fv_hbm.at[idx_v.at[dd, pl.ds(0, ra)]], ft_a, sem_a)

        def gather_b(dd):
            return pltpu.async_copy(
                fv_hbm.at[idx_v.at[dd, pl.ds(ra, rb)]], ft_b, sem_b)

        gather_a(0)

        def d_body(dd, _):
            gather_b(dd)
            pltpu.make_async_copy(
                fv_hbm.at[idx_v.at[dd, pl.ds(0, ra)]], ft_a, sem_a).wait()
            compute(dd, ft_a, 0, ra // _LANES)

            @pl.when(dd + 1 < d)
            def _():
                gather_a(dd + 1)

            pltpu.make_async_copy(
                fv_hbm.at[idx_v.at[dd, pl.ds(ra, rb)]], ft_b, sem_b).wait()
            compute(dd, ft_b, ra, rb // _LANES)
            return 0

        lax.fori_loop(0, d, d_body, 0)
        pltpu.sync_copy(out_v, out_hbm.at[wid])

    return head


def kernel(x, params, ind_from, ind_to):
    f = _backbone(x, params)
    c = f.shape[1]
    hw = f.shape[2] * f.shape[3]
    fv = f[0].reshape(c, hw).T

    nf = ind_from.shape[0]
    d = ind_to.shape[0] // nf
    np_pad = ((nf + 255) // 256) * 256
    c_pad = ((c + 127) // 128) * 128

    nw = 32
    ch = np_pad // nw
    fv = jnp.pad(fv, ((0, 0), (0, c_pad - c)))
    indf_p = jnp.zeros((np_pad,), jnp.int32).at[:nf].set(
        ind_from).reshape(nw, ch)
    indt_p = jnp.zeros((d, np_pad), jnp.int32).at[:, :nf].set(
        ind_to.reshape(d, nf)).reshape(d, nw, ch).transpose(1, 0, 2)

    head = _make_affinity_head(hw, c, c_pad, d, np_pad)
    out = head(fv, indf_p, indt_p)
    out = out.transpose(1, 0, 2).reshape(d, np_pad)
    return out[None, :, :nf]


# speedup vs baseline: 1.0466x; 1.0466x over previous
import functools

import jax
import jax.numpy as jnp
import numpy as np
from jax import lax
from jax.experimental import pallas as pl
from jax.experimental.pallas import tpu as pltpu
from jax.experimental.pallas import tpu_sc as plsc


def _conv(x, w, stride=1, padding=0, dilation=1):
    return lax.conv_general_dilated(
        x, w, (stride, stride), [(padding, padding), (padding, padding)],
        rhs_dilation=(dilation, dilation),
        dimension_numbers=('NCHW', 'OIHW', 'NCHW'))


def _bn(x, p):
    g, b, m, v = p
    inv = g / jnp.sqrt(v + 1e-5)
    return x * inv[None, :, None, None] + (b - m * inv)[None, :, None, None]


def _resblock(x, p, stride=1, first_dilation=1, dilation=1):
    b2 = jax.nn.relu(_bn(x, p['bn2a']))
    xbr = b2
    branch1 = _conv(b2, p['w1'], stride) if 'w1' in p else x
    b2 = _conv(b2, p['w2a'], stride, padding=first_dilation, dilation=first_dilation)
    b2 = jax.nn.relu(_bn(b2, p['bn2b1']))
    b2 = _conv(b2, p['w2b1'], 1, padding=dilation, dilation=dilation)
    return branch1 + b2, xbr


def _resblock_bot(x, p, dilation=1):
    b2 = jax.nn.relu(_bn(x, p['bn2a']))
    xbr = b2
    branch1 = _conv(b2, p['w1'], 1)
    b2 = _conv(b2, p['w2a'], 1)
    b2 = jax.nn.relu(_bn(b2, p['bn2b1']))
    b2 = _conv(b2, p['w2b1'], 1, padding=dilation, dilation=dilation)
    b2 = jax.nn.relu(_bn(b2, p['bn2b2']))
    b2 = _conv(b2, p['w2b2'], 1)
    return branch1 + b2, xbr


def _backbone(x, p):
    h = _conv(x, p['conv1a'], 1, padding=1)
    h, _ = _resblock(h, p['b2'], stride=2)
    h, _ = _resblock(h, p['b2_1'])
    h, _ = _resblock(h, p['b2_2'])
    h, _ = _resblock(h, p['b3'], stride=2)
    h, _ = _resblock(h, p['b3_1'])
    h, _ = _resblock(h, p['b3_2'])
    h, _ = _resblock(h, p['b4'], stride=2)
    for name in ('b4_1', 'b4_2', 'b4_3', 'b4_4', 'b4_5'):
        h, _ = _resblock(h, p[name])
    h, conv4 = _resblock(h, p['b5'], stride=1, first_dilation=1, dilation=2)
    h, _ = _resblock(h, p['b5_1'], dilation=2)
    h, _ = _resblock(h, p['b5_2'], dilation=2)
    h, conv5 = _resblock_bot(h, p['b6'], dilation=4)
    h, _ = _resblock_bot(h, p['b7'], dilation=4)
    conv6 = jax.nn.relu(_bn(h, p['bn7']))
    f8_3 = jax.nn.elu(_conv(conv4, p['f8_3']))
    f8_4 = jax.nn.elu(_conv(conv5, p['f8_4']))
    f8_5 = jax.nn.elu(_conv(conv6, p['f8_5']))
    f = jax.nn.elu(_conv(jnp.concatenate([f8_3, f8_4, f8_5], axis=1), p['f9']))
    return f


_LANES = 16


def _shuffle(v, perm):
    return lax.gather(
        v, perm[:, None],
        dimension_numbers=lax.GatherDimensionNumbers(
            offset_dims=(), collapsed_slice_dims=(0,), start_index_map=(0,)),
        slice_sizes=(1,),
        mode=lax.GatherScatterMode.PROMISE_IN_BOUNDS)


@functools.lru_cache(maxsize=None)
def _make_affinity_head(hw, c, c_pad, d, np_pad):
    nc, ns = 2, 16
    nw = nc * ns
    ch = np_pad // nw
    nch = c // _LANES
    ng = ch // _LANES

    mesh = plsc.VectorSubcoreMesh(core_axis_name="c", subcore_axis_name="s")

    assert d % 2 == 0

    ra = (ng // 2 + 1) * _LANES
    rb = ch - ra

    @functools.partial(
        pl.kernel,
        out_type=jax.ShapeDtypeStruct((nw, d, ch), jnp.float32),
        mesh=mesh,
        scratch_types=[
            pltpu.VMEM((d, ch), jnp.int32),
            pltpu.VMEM((8, ch), jnp.int32),
            pltpu.VMEM((ch, c_pad), jnp.float32),
            pltpu.VMEM((ra, c_pad), jnp.float32),
            pltpu.VMEM((rb, c_pad), jnp.float32),
            pltpu.VMEM((d, ch), jnp.float32),
            pltpu.SemaphoreType.DMA,
            pltpu.SemaphoreType.DMA,
        ],
    )
    def head(fv_hbm, indf_hbm, indt_hbm, out_hbm,
             idx_v, fidx_v, ff_v, ft_a, ft_b, out_v, sem_a, sem_b):
        wid = lax.axis_index("s") * nc + lax.axis_index("c")
        pltpu.sync_copy(indt_hbm.at[wid], idx_v)
        pltpu.sync_copy(indf_hbm.at[wid], fidx_v.at[0])
        pltpu.async_copy(fv_hbm.at[fidx_v.at[0]], ff_v, sem_a).wait()
        lane = lax.iota(jnp.int32, _LANES)
        inv_c = jnp.float32(-1.0 / c)
        perms = [lane ^ jnp.int32(sh) for sh in (8, 4, 2, 1)]

        def compute(dd, ft_v, row0, ngrp):
            def g_body(g, _):
                def r_body(r, rowsums):
                    row = g * _LANES + r
                    accs = [jnp.zeros((_LANES,), jnp.float32) for _ in range(4)]
                    for j in range(nch):
                        t = ft_v[row, pl.ds(j * _LANES, _LANES)]
                        f = ff_v[row0 + row, pl.ds(j * _LANES, _LANES)]
                        accs[j % 4] = accs[j % 4] + jnp.abs(t - f)
                    v = (accs[0] + accs[1]) + (accs[2] + accs[3])
                    for perm in perms:
                        v = v + _shuffle(v, perm)
                    return jnp.where(lane == r, v, rowsums)

                rowsums = plsc.parallel_loop(
                    0, _LANES, carry=jnp.zeros((_LANES,), jnp.float32))(r_body)
                out_v[dd, pl.ds(row0 + g * _LANES, _LANES)] = (
                    jnp.exp(rowsums * inv_c))
                return 0

            lax.fori_loop(0, ngrp, g_body, 0)

        def gather_a(dd):
            return pltpu.async_copy(
                fv_hbm.at[idx_v.at[dd, pl.ds(0, ra)]], ft_a, sem_a)

        def gather_b(dd):
            return pltpu.async_copy(
                fv_hbm.at[idx_v.at[dd, pl.ds(ra, rb)]], ft_b, sem_b)

        gather_a(0)

        def d_body(dd, _):
            gather_b(dd)
            pltpu.make_async_copy(
                fv_hbm.at[idx_v.at[dd, pl.ds(0, ra)]], ft_a, sem_a).wait()
            compute(dd, ft_a, 0, ra // _LANES)

            @pl.when(dd + 1 < d)
            def _():
                gather_a(dd + 1)

            pltpu.make_async_copy(
                fv_hbm.at[idx_v.at[dd, pl.ds(ra, rb)]], ft_b, sem_b).wait()
            compute(dd, ft_b, ra, rb // _LANES)
            return 0

        lax.fori_loop(0, d, d_body, 0)
        pltpu.sync_copy(out_v, out_hbm.at[wid])

    return head


def kernel(x, params, ind_from, ind_to):
    f = _backbone(x, params)
    c = f.shape[1]
    hw = f.shape[2] * f.shape[3]
    fv = f[0].reshape(c, hw).T

    nf = ind_from.shape[0]
    d = ind_to.shape[0] // nf
    np_pad = ((nf + 255) // 256) * 256
    c_pad = ((c + 127) // 128) * 128

    nw = 32
    ch = np_pad // nw
    fv = jnp.pad(fv, ((0, 0), (0, c_pad - c)))
    indf_p = jnp.zeros((np_pad,), jnp.int32).at[:nf].set(
        ind_from).reshape(nw, ch)
    indt_p = jnp.zeros((d, np_pad), jnp.int32).at[:, :nf].set(
        ind_to.reshape(d, nf)).reshape(d, nw, ch).transpose(1, 0, 2)

    head = _make_affinity_head(hw, c, c_pad, d, np_pad)
    out = head(fv, indf_p, indt_p)
    out = out.transpose(1, 0, 2).reshape(d, np_pad)
    return out[None, :, :nf]
